# baseline (speedup 1.0000x reference)
import functools
import math

import jax
import jax.numpy as jnp
from jax.experimental import pallas as pl
from jax.experimental.pallas import tpu as pltpu

_DELTA = 0.1
_SIGMA = 0.1
_LOG2E = math.log2(math.e)
_BETA = 0.5 * _LOG2E / (_DELTA * _DELTA)
_TWOB = 2.0 * _BETA
_GAMMA = _LOG2E / _SIGMA


def _dot_bf16(a, b):
    return jax.lax.dot_general(
        a.astype(jnp.bfloat16), b.astype(jnp.bfloat16),
        dimension_numbers=(((1,), (0,)), ((), ())),
        preferred_element_type=jnp.float32)


def _nms_tile_kernel(as_ref, at_ref, bs_ref, bt_ref,
                     ssc_ref, ssr_ref, ttc_ref, ttr_ref,
                     srow_ref, scol_ref, out_ref, *, nc):
    j = pl.program_id(1)
    ms = _dot_bf16(as_ref[...], bs_ref[...])
    mt = _dot_bf16(at_ref[...], bt_ref[...])
    es = jnp.minimum(_TWOB * ms - (ssc_ref[...] + ssr_ref[...]), 0.0)
    et = jnp.minimum(_TWOB * mt - (ttc_ref[...] + ttr_ref[...]), 0.0)
    o = jnp.exp2(es + et)
    mask = srow_ref[...] > scol_ref[...]
    p = jnp.where(mask, o, 0.0)
    psum = jnp.sum(p, axis=1, keepdims=True)
    acc = jnp.where(j == 0, 0.0, out_ref[...]) + psum
    out_ref[...] = jnp.where(j == nc - 1,
                             scol_ref[...] * jnp.exp2(-_GAMMA * acc),
                             acc)


def kernel(src_points, tgt_points, scores):
    n = scores.shape[0]
    R, C = 256, 1024
    tile = max(R, C)
    n_pad = ((n + tile - 1) // tile) * tile

    xs = src_points.astype(jnp.float32)
    xt = tgt_points.astype(jnp.float32)
    sqs = _BETA * jnp.sum(xs * xs, axis=1, keepdims=True)
    sqt = _BETA * jnp.sum(xt * xt, axis=1, keepdims=True)

    def pad8(x):
        return jnp.pad(x, ((0, n_pad - n), (0, 8 - x.shape[1])))

    a_s, a_t = pad8(xs), pad8(xt)
    b_s, b_t = a_s.T, a_t.T
    padv = ((0, n_pad - n), (0, 0))
    ssc = jnp.pad(sqs, padv)
    ttc = jnp.pad(sqt, padv)
    ssr, ttr = ssc.T, ttc.T
    s_pad = jnp.pad(scores.astype(jnp.float32), (0, n_pad - n),
                    constant_values=-jnp.inf)
    srow = s_pad[None, :]
    scol = s_pad[:, None]

    grid = (n_pad // R, n_pad // C)
    row = lambda i, j: (i, 0)
    col = lambda i, j: (0, j)
    out = pl.pallas_call(
        functools.partial(_nms_tile_kernel, nc=grid[1]),
        grid=grid,
        in_specs=[
            pl.BlockSpec((R, 8), row),
            pl.BlockSpec((R, 8), row),
            pl.BlockSpec((8, C), col),
            pl.BlockSpec((8, C), col),
            pl.BlockSpec((R, 1), row),
            pl.BlockSpec((1, C), col),
            pl.BlockSpec((R, 1), row),
            pl.BlockSpec((1, C), col),
            pl.BlockSpec((1, C), col),
            pl.BlockSpec((R, 1), row),
        ],
        out_specs=pl.BlockSpec((R, 1), row),
        out_shape=jax.ShapeDtypeStruct((n_pad, 1), jnp.float32),
        compiler_params=pltpu.CompilerParams(
            dimension_semantics=("parallel", "arbitrary")),
    )(a_s, a_t, b_s, b_t, ssc, ssr, ttc, ttr, srow, scol)
    return out[:n, 0]
